# baseline (speedup 1.0000x reference)
import jax
import jax.numpy as jnp
from jax import lax
from jax.experimental import pallas as pl
from jax.experimental.pallas import tpu as pltpu

B = 128
N = 5000
E = 64
N2 = N * E // 128


def _body(uid_ref, iid_ref, uv_ref, iv_ref, umem, imem,
          wih_u_ref, whh_u_ref, bih_u_ref, bhh_u_ref,
          wih_i_ref, whh_i_ref, bih_i_ref, bhh_i_ref,
          out_umem, out_imem, new_u3, new_i3,
          um_f, im_f, pu_s, pi_s, g_sem, s_sem):
    def g_start(b, _):
        pltpu.make_async_copy(
            umem.at[pl.ds(b, 1), pl.ds(uid_ref[b] // 2, 1)],
            um_f.at[pl.ds(b, 1)], g_sem).start()
        pltpu.make_async_copy(
            imem.at[pl.ds(b, 1), pl.ds(iid_ref[b] // 2, 1)],
            im_f.at[pl.ds(b, 1)], g_sem).start()
        return 0
    lax.fori_loop(0, B, g_start, 0)
    pltpu.make_async_copy(um_f, um_f, g_sem).wait()
    pltpu.make_async_copy(im_f, im_f, g_sem).wait()

    hu = (uv_ref[...] % 2) * E
    hi = (iv_ref[...] % 2) * E
    umf = um_f[:, 0, :]
    imf = im_f[:, 0, :]
    lane = lax.broadcasted_iota(jnp.int32, (1, 128), 1)
    um = jnp.where(hu == 0, umf[:, :E], umf[:, E:])
    im = jnp.where(hi == 0, imf[:, :E], imf[:, E:])

    x_u = jnp.concatenate([um, im], axis=1)
    x_i = jnp.concatenate([im, um], axis=1)

    def cell(x, h, wih, whh, bih, bhh):
        gi = lax.dot_general(x, wih, (((1,), (1,)), ((), ())),
                             preferred_element_type=jnp.float32) + bih
        gh = lax.dot_general(h, whh, (((1,), (1,)), ((), ())),
                             preferred_element_type=jnp.float32) + bhh
        i_r, i_z, i_n = gi[:, :E], gi[:, E:2 * E], gi[:, 2 * E:]
        h_r, h_z, h_n = gh[:, :E], gh[:, E:2 * E], gh[:, 2 * E:]
        r = jax.nn.sigmoid(i_r + h_r)
        z = jax.nn.sigmoid(i_z + h_z)
        n = jnp.tanh(i_n + r * h_n)
        h2 = (1.0 - z) * n + z * h
        nrm = jnp.sqrt(jnp.sum(h2 * h2, axis=1, keepdims=True))
        return h2 / jnp.maximum(nrm, 1e-12)

    new_u = cell(x_u, um, wih_u_ref[...], whh_u_ref[...],
                 bih_u_ref[...], bhh_u_ref[...])
    new_i = cell(x_i, im, wih_i_ref[...], whh_i_ref[...],
                 bih_i_ref[...], bhh_i_ref[...])
    new_u3[:, 0, :] = new_u
    new_i3[:, 0, :] = new_i

    mu = (lane >= hu) & (lane < hu + E)
    mi = (lane >= hi) & (lane < hi + E)
    pu_s[:, 0, :] = jnp.where(mu, jnp.concatenate([new_u, new_u], axis=1),
                              umf)
    pi_s[:, 0, :] = jnp.where(mi, jnp.concatenate([new_i, new_i], axis=1),
                              imf)

    def s_start(b, _):
        pltpu.make_async_copy(
            pu_s.at[pl.ds(b, 1)],
            out_umem.at[pl.ds(b, 1), pl.ds(uid_ref[b] // 2, 1)],
            s_sem).start()
        pltpu.make_async_copy(
            pi_s.at[pl.ds(b, 1)],
            out_imem.at[pl.ds(b, 1), pl.ds(iid_ref[b] // 2, 1)],
            s_sem).start()
        return 0
    lax.fori_loop(0, B, s_start, 0)
    pltpu.make_async_copy(pu_s, pu_s, s_sem).wait()
    pltpu.make_async_copy(pi_s, pi_s, s_sem).wait()


def kernel(user_ids, item_ids, user_features, item_features,
           user_memory, item_memory,
           Wih_u, Whh_u, bih_u, bhh_u, Wih_i, Whh_i, bih_i, bhh_i):
    uid = user_ids.astype(jnp.int32)
    iid = item_ids.astype(jnp.int32)
    u2 = user_memory.reshape(B, N2, 128)
    i2 = item_memory.reshape(B, N2, 128)

    smem = pl.BlockSpec(memory_space=pltpu.SMEM)
    anym = pl.BlockSpec(memory_space=pl.ANY)
    vmem = pl.BlockSpec(memory_space=pltpu.VMEM)

    uo, io, new_u3, new_i3 = pl.pallas_call(
        _body,
        in_specs=[smem, smem, vmem, vmem, anym, anym,
                  vmem, vmem, vmem, vmem, vmem, vmem, vmem, vmem],
        out_specs=[anym, anym, vmem, vmem],
        out_shape=[
            jax.ShapeDtypeStruct((B, N2, 128), jnp.float32),
            jax.ShapeDtypeStruct((B, N2, 128), jnp.float32),
            jax.ShapeDtypeStruct((B, 1, E), jnp.float32),
            jax.ShapeDtypeStruct((B, 1, E), jnp.float32),
        ],
        scratch_shapes=[
            pltpu.VMEM((B, 1, 128), jnp.float32),
            pltpu.VMEM((B, 1, 128), jnp.float32),
            pltpu.VMEM((B, 1, 128), jnp.float32),
            pltpu.VMEM((B, 1, 128), jnp.float32),
            pltpu.SemaphoreType.DMA,
            pltpu.SemaphoreType.DMA,
        ],
        input_output_aliases={4: 0, 5: 1},
    )(uid, iid, uid.reshape(B, 1), iid.reshape(B, 1), u2, i2,
      Wih_u, Whh_u, bih_u.reshape(1, 3 * E), bhh_u.reshape(1, 3 * E),
      Wih_i, Whh_i, bih_i.reshape(1, 3 * E), bhh_i.reshape(1, 3 * E))

    new_u = new_u3.reshape(B, E)
    new_i = new_i3.reshape(B, E)
    out = jnp.concatenate([
        user_ids[:, None].astype(jnp.float32),
        item_ids[:, None].astype(jnp.float32),
        new_u,
        new_i,
    ], axis=1)
    return out, uo.reshape(B, N, E), io.reshape(B, N, E)
